# baseline (speedup 1.0000x reference)
import jax
import jax.numpy as jnp
from jax.experimental import pallas as pl
from jax.experimental.pallas import tpu as pltpu


def _sigmoid(x):
    return 0.5 * jnp.tanh(0.5 * x) + 0.5


def _act_body(xyz_in, rgb_ref, opa_ref, scl_ref,
              xyz_out, rgb_out, opa_out, scl_out, sem):
    i = pl.program_id(0)

    @pl.when(i == 0)
    def _start():
        pltpu.make_async_copy(xyz_in, xyz_out, sem).start()

    rgb_out[...] = _sigmoid(rgb_ref[...])
    opa_out[...] = _sigmoid(opa_ref[...])
    scl_out[...] = jnp.exp(scl_ref[...])

    @pl.when(i == pl.num_programs(0) - 1)
    def _finish():
        pltpu.make_async_copy(xyz_in, xyz_out, sem).wait()


def kernel(xyz_raw, rgb_raw, opacity_raw, scale_raw):
    n = rgb_raw.shape[0]
    xyzT = xyz_raw.T
    rgbT = rgb_raw.T
    opaT = opacity_raw.T
    sclT = scale_raw.T

    C = 65536
    grid = (pl.cdiv(n, C),)
    xyz_o, rgb_o, opa_o, scl_o = pl.pallas_call(
        _act_body,
        grid=grid,
        in_specs=[
            pl.BlockSpec(memory_space=pl.ANY),
            pl.BlockSpec((3, C), lambda i: (0, i)),
            pl.BlockSpec((1, C), lambda i: (0, i)),
            pl.BlockSpec((1, C), lambda i: (0, i)),
        ],
        out_specs=[
            pl.BlockSpec(memory_space=pl.ANY),
            pl.BlockSpec((3, C), lambda i: (0, i)),
            pl.BlockSpec((1, C), lambda i: (0, i)),
            pl.BlockSpec((1, C), lambda i: (0, i)),
        ],
        out_shape=[
            jax.ShapeDtypeStruct((3, n), jnp.float32),
            jax.ShapeDtypeStruct((3, n), jnp.float32),
            jax.ShapeDtypeStruct((1, n), jnp.float32),
            jax.ShapeDtypeStruct((1, n), jnp.float32),
        ],
        scratch_shapes=[pltpu.SemaphoreType.DMA],
        compiler_params=pltpu.CompilerParams(
            dimension_semantics=("arbitrary",),
        ),
    )(xyzT, rgbT, opaT, sclT)

    return (
        xyz_o.T,
        rgb_o.T,
        opa_o.T,
        scl_o.T,
    )


# speedup vs baseline: 16.1613x; 16.1613x over previous
import jax
import jax.numpy as jnp
from jax.experimental import pallas as pl
from jax.experimental.pallas import tpu as pltpu


def _sigmoid(x):
    return 0.5 * jnp.tanh(0.5 * x) + 0.5


def _act_body(rgb_ref, opa_ref, scl_ref,
              rgb_out, opa_out, scl_out):
    rgb_out[...] = _sigmoid(rgb_ref[...])
    opa_out[...] = _sigmoid(opa_ref[...])
    scl_out[...] = jnp.exp(scl_ref[...])


def kernel(xyz_raw, rgb_raw, opacity_raw, scale_raw):
    n = rgb_raw.shape[0]
    rgbT = rgb_raw.T
    opaT = opacity_raw.T
    sclT = scale_raw.T

    C = 65536
    grid = (pl.cdiv(n, C),)
    rgb_o, opa_o, scl_o = pl.pallas_call(
        _act_body,
        grid=grid,
        in_specs=[
            pl.BlockSpec((3, C), lambda i: (0, i)),
            pl.BlockSpec((1, C), lambda i: (0, i)),
            pl.BlockSpec((1, C), lambda i: (0, i)),
        ],
        out_specs=[
            pl.BlockSpec((3, C), lambda i: (0, i)),
            pl.BlockSpec((1, C), lambda i: (0, i)),
            pl.BlockSpec((1, C), lambda i: (0, i)),
        ],
        out_shape=[
            jax.ShapeDtypeStruct((3, n), jnp.float32),
            jax.ShapeDtypeStruct((1, n), jnp.float32),
            jax.ShapeDtypeStruct((1, n), jnp.float32),
        ],
        compiler_params=pltpu.CompilerParams(
            dimension_semantics=("arbitrary",),
        ),
    )(rgbT, opaT, sclT)

    return (
        xyz_raw,
        rgb_o.T,
        opa_o.T,
        scl_o.T,
    )


# speedup vs baseline: 17.8224x; 1.1028x over previous
import jax
import jax.numpy as jnp
from jax.experimental import pallas as pl
from jax.experimental.pallas import tpu as pltpu


def _sigmoid(x):
    return 0.5 * jnp.tanh(0.5 * x) + 0.5


def _act_body(xyz_ref, rgb_ref, opa_ref, scl_ref,
           xyz_out, rgb_out, opa_out, scl_out):
    xyz_out[...] = xyz_ref[...]
    rgb_out[...] = _sigmoid(rgb_ref[...])
    opa_out[...] = _sigmoid(opa_ref[...])
    scl_out[...] = jnp.exp(scl_ref[...])


def kernel(xyz_raw, rgb_raw, opacity_raw, scale_raw):
    n = rgb_raw.shape[0]
    xyzT = xyz_raw.T
    rgbT = rgb_raw.T
    opaT = opacity_raw.T
    sclT = scale_raw.T

    C = 65536
    grid = (pl.cdiv(n, C),)
    s3 = pl.BlockSpec((3, C), lambda i: (0, i))
    s1 = pl.BlockSpec((1, C), lambda i: (0, i))
    xyz_o, rgb_o, opa_o, scl_o = pl.pallas_call(
        _act_body,
        grid=grid,
        in_specs=[s3, s3, s1, s1],
        out_specs=[s3, s3, s1, s1],
        out_shape=[
            jax.ShapeDtypeStruct((3, n), jnp.float32),
            jax.ShapeDtypeStruct((3, n), jnp.float32),
            jax.ShapeDtypeStruct((1, n), jnp.float32),
            jax.ShapeDtypeStruct((1, n), jnp.float32),
        ],
        compiler_params=pltpu.CompilerParams(
            dimension_semantics=("arbitrary",),
        ),
    )(xyzT, rgbT, opaT, sclT)

    return (xyz_o.T, rgb_o.T, opa_o.T, scl_o.T)


# speedup vs baseline: 19.0401x; 1.0683x over previous
import jax
import jax.numpy as jnp
from jax.experimental import pallas as pl
from jax.experimental.pallas import tpu as pltpu


def _sigmoid(x):
    return 0.5 * jnp.tanh(0.5 * x) + 0.5


def _act_body(xyz_ref, rgb_ref, opa_ref, scl_ref,
           xyz_out, rgb_out, opa_out, scl_out):
    xyz_out[...] = xyz_ref[...]
    rgb_out[...] = _sigmoid(rgb_ref[...])
    opa_out[...] = _sigmoid(opa_ref[...])
    scl_out[...] = jnp.exp(scl_ref[...])


def kernel(xyz_raw, rgb_raw, opacity_raw, scale_raw):
    n = rgb_raw.shape[0]
    xyzT = xyz_raw.T
    rgbT = rgb_raw.T
    opaT = opacity_raw.T
    sclT = scale_raw.T

    C = 131072
    grid = (pl.cdiv(n, C),)
    s3 = pl.BlockSpec((3, C), lambda i: (0, i))
    s1 = pl.BlockSpec((1, C), lambda i: (0, i))
    xyz_o, rgb_o, opa_o, scl_o = pl.pallas_call(
        _act_body,
        grid=grid,
        in_specs=[s3, s3, s1, s1],
        out_specs=[s3, s3, s1, s1],
        out_shape=[
            jax.ShapeDtypeStruct((3, n), jnp.float32),
            jax.ShapeDtypeStruct((3, n), jnp.float32),
            jax.ShapeDtypeStruct((1, n), jnp.float32),
            jax.ShapeDtypeStruct((1, n), jnp.float32),
        ],
        compiler_params=pltpu.CompilerParams(
            dimension_semantics=("arbitrary",),
        ),
    )(xyzT, rgbT, opaT, sclT)

    return (xyz_o.T, rgb_o.T, opa_o.T, scl_o.T)


# speedup vs baseline: 19.6356x; 1.0313x over previous
import jax
import jax.numpy as jnp
from jax.experimental import pallas as pl
from jax.experimental.pallas import tpu as pltpu


def _sigmoid(x):
    return 0.5 * jnp.tanh(0.5 * x) + 0.5


def _act_body(xyz_ref, rgb_ref, opa_ref, scl_ref,
           xyz_out, rgb_out, opa_out, scl_out):
    xyz_out[...] = xyz_ref[...]
    rgb_out[...] = _sigmoid(rgb_ref[...])
    opa_out[...] = _sigmoid(opa_ref[...])
    scl_out[...] = jnp.exp(scl_ref[...])


def kernel(xyz_raw, rgb_raw, opacity_raw, scale_raw):
    n = rgb_raw.shape[0]
    xyzT = xyz_raw.T
    rgbT = rgb_raw.T
    opaT = opacity_raw.T
    sclT = scale_raw.T

    C = 262144
    grid = (pl.cdiv(n, C),)
    s3 = pl.BlockSpec((3, C), lambda i: (0, i))
    s1 = pl.BlockSpec((1, C), lambda i: (0, i))
    xyz_o, rgb_o, opa_o, scl_o = pl.pallas_call(
        _act_body,
        grid=grid,
        in_specs=[s3, s3, s1, s1],
        out_specs=[s3, s3, s1, s1],
        out_shape=[
            jax.ShapeDtypeStruct((3, n), jnp.float32),
            jax.ShapeDtypeStruct((3, n), jnp.float32),
            jax.ShapeDtypeStruct((1, n), jnp.float32),
            jax.ShapeDtypeStruct((1, n), jnp.float32),
        ],
        compiler_params=pltpu.CompilerParams(
            dimension_semantics=("arbitrary",),
        ),
    )(xyzT, rgbT, opaT, sclT)

    return (xyz_o.T, rgb_o.T, opa_o.T, scl_o.T)


# speedup vs baseline: 19.7903x; 1.0079x over previous
import jax
import jax.numpy as jnp
from jax.experimental import pallas as pl
from jax.experimental.pallas import tpu as pltpu


def _sigmoid(x):
    return 0.5 * jnp.tanh(0.5 * x) + 0.5


def _act_body(xyz_ref, rgb_ref, opa_ref, scl_ref,
           xyz_out, rgb_out, opa_out, scl_out):
    xyz_out[...] = xyz_ref[...]
    rgb_out[...] = _sigmoid(rgb_ref[...])
    opa_out[...] = _sigmoid(opa_ref[...])
    scl_out[...] = jnp.exp(scl_ref[...])


def kernel(xyz_raw, rgb_raw, opacity_raw, scale_raw):
    n = rgb_raw.shape[0]
    xyzT = xyz_raw.T
    rgbT = rgb_raw.T
    opaT = opacity_raw.T
    sclT = scale_raw.T

    C = 327680
    grid = (pl.cdiv(n, C),)
    s3 = pl.BlockSpec((3, C), lambda i: (0, i))
    s1 = pl.BlockSpec((1, C), lambda i: (0, i))
    xyz_o, rgb_o, opa_o, scl_o = pl.pallas_call(
        _act_body,
        grid=grid,
        in_specs=[s3, s3, s1, s1],
        out_specs=[s3, s3, s1, s1],
        out_shape=[
            jax.ShapeDtypeStruct((3, n), jnp.float32),
            jax.ShapeDtypeStruct((3, n), jnp.float32),
            jax.ShapeDtypeStruct((1, n), jnp.float32),
            jax.ShapeDtypeStruct((1, n), jnp.float32),
        ],
        compiler_params=pltpu.CompilerParams(
            dimension_semantics=("arbitrary",),
        ),
    )(xyzT, rgbT, opaT, sclT)

    return (xyz_o.T, rgb_o.T, opa_o.T, scl_o.T)
